# baseline (speedup 1.0000x reference)
import functools

import jax
import jax.numpy as jnp
from jax import lax
from jax.experimental import pallas as pl
from jax.experimental.pallas import tpu as pltpu
from jax.experimental.pallas import tpu_sc as plsc

_NUM_WORKERS = 32
_CHUNK = 200


def kernel(V_set, E_set, node_ids):
    V = V_set[0]
    M, De = E_set.shape[1], E_set.shape[2]
    D = V.shape[1]
    E128 = E_set.reshape(M * De // 128, 128)
    idx = node_ids[0].astype(jnp.int32)
    b_per_w = M // _NUM_WORKERS
    n_chunks = b_per_w // _CHUNK
    n_pairs = n_chunks // 2
    erows = _CHUNK * De // 128
    epack = 128 // De

    mesh = plsc.VectorSubcoreMesh(core_axis_name="c", subcore_axis_name="s")

    @functools.partial(
        pl.kernel,
        mesh=mesh,
        out_type=jax.ShapeDtypeStruct((M, De + D), jnp.float32),
        scratch_types=[
            pltpu.VMEM((_CHUNK,), jnp.int32),
            pltpu.VMEM((_CHUNK,), jnp.int32),
            pltpu.VMEM((_CHUNK, D), jnp.float32),
            pltpu.VMEM((_CHUNK, D), jnp.float32),
            pltpu.VMEM((_CHUNK * 16 // 128, 128), jnp.float32),
            pltpu.VMEM((_CHUNK * 16 // 128, 128), jnp.float32),
            pltpu.VMEM((_CHUNK, 16), jnp.float32),
            pltpu.VMEM((_CHUNK, 16), jnp.float32),
            pltpu.SemaphoreType.DMA,
            pltpu.SemaphoreType.DMA,
            pltpu.SemaphoreType.DMA,
            pltpu.SemaphoreType.DMA,
            pltpu.SemaphoreType.DMA,
            pltpu.SemaphoreType.DMA,
        ],
        compiler_params=pltpu.CompilerParams(use_tc_tiling_on_sc=False),
    )
    def _k(v_hbm, e_hbm, idx_hbm, out_hbm, idx0, idx1, rows0, rows1,
           ev0, ev1, ec0, ec1, g0, g1, o0, o1, se0, se1):
        wid = lax.axis_index("s") * 2 + lax.axis_index("c")
        base = wid * b_per_w
        idxs = (idx0, idx1)
        rows = (rows0, rows1)
        evs = (ev0, ev1)
        ecombs = (ec0, ec1)
        sg = (g0, g1)
        so = (o0, o1)
        se = (se0, se1)

        def gather_start(c, b):
            pltpu.sync_copy(idx_hbm.at[pl.ds(base + c * _CHUNK, _CHUNK)],
                            idxs[b])
            pltpu.make_async_copy(v_hbm.at[idxs[b]], rows[b], sg[b]).start()
            pltpu.make_async_copy(
                e_hbm.at[pl.ds((base + c * _CHUNK) * De // 128, erows)],
                evs[b], se[b]).start()

        def e_wait(b):
            pltpu.make_async_copy(
                e_hbm.at[pl.ds(0, erows)], evs[b], se[b]).wait()

        def v_wait(b):
            pltpu.make_async_copy(v_hbm.at[idxs[b]], rows[b], sg[b]).wait()

        def fill_e(b):
            def rbody(rr, carry):
                for k in range(epack):
                    ecombs[b][rr * epack + k, pl.ds(0, De)] = (
                        evs[b][rr, pl.ds(k * De, De)])
                return carry
            lax.fori_loop(0, erows, rbody, 0)

        def out_start(c, b):
            pltpu.make_async_copy(
                rows[b],
                out_hbm.at[pl.ds(base + c * _CHUNK, _CHUNK), pl.ds(De, D)],
                so[b]).start()
            pltpu.make_async_copy(
                ecombs[b],
                out_hbm.at[pl.ds(base + c * _CHUNK, _CHUNK), pl.ds(0, De)],
                so[b]).start()

        def out_wait(b):
            pltpu.make_async_copy(
                rows[b],
                out_hbm.at[pl.ds(0, _CHUNK), pl.ds(De, D)], so[b]).wait()
            pltpu.make_async_copy(
                ecombs[b],
                out_hbm.at[pl.ds(0, _CHUNK), pl.ds(0, De)], so[b]).wait()

        gather_start(0, 0)

        def body(c2, carry):
            for b in (0, 1):
                c = 2 * c2 + b
                e_wait(b)
                fill_e(b)
                v_wait(b)
                out_start(c, b)
                nb = 1 - b
                if b == 0:
                    @pl.when(c2 >= 1)
                    def _():
                        out_wait(nb)
                    gather_start(c + 1, nb)
                else:
                    @pl.when(c2 < n_pairs - 1)
                    def _():
                        out_wait(nb)
                        gather_start(c + 1, nb)
            return carry

        lax.fori_loop(0, n_pairs, body, 0)

        out_wait(0)
        out_wait(1)

    out = _k(V, E128, idx)
    return out[jnp.newaxis]


# speedup vs baseline: 2.5091x; 2.5091x over previous
import functools

import jax
import jax.numpy as jnp
from jax import lax
from jax.experimental import pallas as pl
from jax.experimental.pallas import tpu as pltpu
from jax.experimental.pallas import tpu_sc as plsc

_NUM_WORKERS = 32
_CHUNK = 200
_BE = 3200


def kernel(V_set, E_set, node_ids):
    V = V_set[0]
    M, De = E_set.shape[1], E_set.shape[2]
    D = V.shape[1]
    E_t = E_set[0].T
    idx = node_ids[0].astype(jnp.int32)
    b_per_w = M // _NUM_WORKERS
    n_chunks = b_per_w // _CHUNK
    n_pairs = n_chunks // 2

    mesh = plsc.VectorSubcoreMesh(core_axis_name="c", subcore_axis_name="s")

    @functools.partial(
        pl.kernel,
        mesh=mesh,
        out_type=jax.ShapeDtypeStruct((M, D), jnp.float32),
        scratch_types=[
            pltpu.VMEM((_CHUNK,), jnp.int32),
            pltpu.VMEM((_CHUNK,), jnp.int32),
            pltpu.VMEM((_CHUNK, D), jnp.float32),
            pltpu.VMEM((_CHUNK, D), jnp.float32),
            pltpu.SemaphoreType.DMA,
            pltpu.SemaphoreType.DMA,
            pltpu.SemaphoreType.DMA,
            pltpu.SemaphoreType.DMA,
        ],
        compiler_params=pltpu.CompilerParams(use_tc_tiling_on_sc=False),
    )
    def _gather(v_hbm, idx_hbm, g_hbm, idx0, idx1, rows0, rows1,
                g0, g1, o0, o1):
        wid = lax.axis_index("s") * 2 + lax.axis_index("c")
        base = wid * b_per_w
        idxs = (idx0, idx1)
        rows = (rows0, rows1)
        sg = (g0, g1)
        so = (o0, o1)

        def gather_start(c, b):
            pltpu.sync_copy(idx_hbm.at[pl.ds(base + c * _CHUNK, _CHUNK)],
                            idxs[b])
            pltpu.make_async_copy(v_hbm.at[idxs[b]], rows[b], sg[b]).start()

        def gather_wait(b):
            pltpu.make_async_copy(v_hbm.at[idxs[b]], rows[b], sg[b]).wait()

        def out_start(c, b):
            pltpu.make_async_copy(
                rows[b], g_hbm.at[pl.ds(base + c * _CHUNK, _CHUNK)],
                so[b]).start()

        def out_wait(b):
            pltpu.make_async_copy(
                rows[b], g_hbm.at[pl.ds(0, _CHUNK)], so[b]).wait()

        gather_start(0, 0)

        def body(c2, carry):
            for b in (0, 1):
                c = 2 * c2 + b
                gather_wait(b)
                out_start(c, b)
                nb = 1 - b
                if b == 0:
                    @pl.when(c2 >= 1)
                    def _():
                        out_wait(nb)
                    gather_start(c + 1, nb)
                else:
                    @pl.when(c2 < n_pairs - 1)
                    def _():
                        out_wait(nb)
                        gather_start(c + 1, nb)
            return carry

        lax.fori_loop(0, n_pairs, body, 0)
        out_wait(0)
        out_wait(1)

    G = _gather(V, idx)

    def _concat(e_ref, g_ref, o_ref):
        o_ref[0:De, :] = e_ref[...]
        o_ref[De:De + D, :] = g_ref[...].T

    out_t = pl.pallas_call(
        _concat,
        grid=(M // _BE,),
        in_specs=[
            pl.BlockSpec((De, _BE), lambda i: (0, i)),
            pl.BlockSpec((_BE, D), lambda i: (i, 0)),
        ],
        out_specs=pl.BlockSpec((De + D, _BE), lambda i: (0, i)),
        out_shape=jax.ShapeDtypeStruct((De + D, M), jnp.float32),
    )(E_t, G)

    return out_t.T[jnp.newaxis]


# speedup vs baseline: 2.6938x; 1.0736x over previous
import functools

import jax
import jax.numpy as jnp
from jax import lax
from jax.experimental import pallas as pl
from jax.experimental.pallas import tpu as pltpu
from jax.experimental.pallas import tpu_sc as plsc

_NUM_WORKERS = 32
_NUM_SEG = 2
_CHUNK = 200
_BE = 3200


def kernel(V_set, E_set, node_ids):
    V = V_set[0]
    M, De = E_set.shape[1], E_set.shape[2]
    D = V.shape[1]
    E_t = E_set[0].T
    idx = node_ids[0].astype(jnp.int32)
    segM = M // _NUM_SEG
    b_per_w = segM // _NUM_WORKERS
    n_chunks = b_per_w // _CHUNK
    n_pairs = n_chunks // 2
    nb_seg = segM // _BE

    mesh = plsc.VectorSubcoreMesh(core_axis_name="c", subcore_axis_name="s")

    def make_gather(seg):
        @functools.partial(
            pl.kernel,
            mesh=mesh,
            out_type=jax.ShapeDtypeStruct((segM, D), jnp.float32),
            scratch_types=[
                pltpu.VMEM((_CHUNK,), jnp.int32),
                pltpu.VMEM((_CHUNK,), jnp.int32),
                pltpu.VMEM((_CHUNK, D), jnp.float32),
                pltpu.VMEM((_CHUNK, D), jnp.float32),
                pltpu.SemaphoreType.DMA,
                pltpu.SemaphoreType.DMA,
                pltpu.SemaphoreType.DMA,
                pltpu.SemaphoreType.DMA,
            ],
            compiler_params=pltpu.CompilerParams(use_tc_tiling_on_sc=False),
        )
        def _gather(v_hbm, idx_hbm, g_hbm, idx0, idx1, rows0, rows1,
                    g0, g1, o0, o1):
            wid = lax.axis_index("s") * 2 + lax.axis_index("c")
            base = seg * segM + wid * b_per_w
            obase = wid * b_per_w
            idxs = (idx0, idx1)
            rows = (rows0, rows1)
            sg = (g0, g1)
            so = (o0, o1)

            def gather_start(c, b):
                pltpu.sync_copy(idx_hbm.at[pl.ds(base + c * _CHUNK, _CHUNK)],
                                idxs[b])
                pltpu.make_async_copy(v_hbm.at[idxs[b]], rows[b],
                                      sg[b]).start()

            def gather_wait(b):
                pltpu.make_async_copy(v_hbm.at[idxs[b]], rows[b],
                                      sg[b]).wait()

            def out_start(c, b):
                pltpu.make_async_copy(
                    rows[b], g_hbm.at[pl.ds(obase + c * _CHUNK, _CHUNK)],
                    so[b]).start()

            def out_wait(b):
                pltpu.make_async_copy(
                    rows[b], g_hbm.at[pl.ds(0, _CHUNK)], so[b]).wait()

            gather_start(0, 0)

            def body(c2, carry):
                for b in (0, 1):
                    c = 2 * c2 + b
                    gather_wait(b)
                    out_start(c, b)
                    nb = 1 - b
                    if b == 0:
                        @pl.when(c2 >= 1)
                        def _():
                            out_wait(nb)
                        gather_start(c + 1, nb)
                    elif n_chunks % 2 == 1:
                        out_wait(nb)
                        gather_start(c + 1, nb)
                    else:
                        @pl.when(c2 < n_pairs - 1)
                        def _():
                            out_wait(nb)
                            gather_start(c + 1, nb)
                return carry

            lax.fori_loop(0, n_pairs, body, 0)
            if n_chunks % 2 == 1:
                gather_wait(0)
                out_start(n_chunks - 1, 0)
                out_wait(1)
                out_wait(0)
            else:
                out_wait(0)
                out_wait(1)

        return _gather

    Gs = [make_gather(s)(V, idx) for s in range(_NUM_SEG)]

    out_t = None
    for s in range(_NUM_SEG):
        def _concat(*refs, _s=s):
            e_ref, g_ref = refs[0], refs[1]
            o_ref = refs[-1]
            o_ref[0:De, :] = e_ref[...]
            o_ref[De:De + D, :] = g_ref[...].T

        in_specs = [
            pl.BlockSpec((De, _BE), lambda i, _s=s: (0, i + _s * nb_seg)),
            pl.BlockSpec((_BE, D), lambda i: (i, 0)),
        ]
        operands = [E_t, Gs[s]]
        aliases = {}
        if s > 0:
            in_specs.append(pl.BlockSpec(memory_space=pl.ANY))
            operands.append(out_t)
            aliases = {2: 0}
        out_t = pl.pallas_call(
            _concat,
            grid=(nb_seg,),
            in_specs=in_specs,
            out_specs=pl.BlockSpec((De + D, _BE),
                                   lambda i, _s=s: (0, i + _s * nb_seg)),
            out_shape=jax.ShapeDtypeStruct((De + D, M), jnp.float32),
            input_output_aliases=aliases,
        )(*operands)

    return out_t.T[jnp.newaxis]
